# baseline (speedup 1.0000x reference)
import functools

import jax
import jax.numpy as jnp
from jax import lax
from jax.experimental import pallas as pl
from jax.experimental.pallas import tpu as pltpu
from jax.experimental.pallas import tpu_sc as plsc

VOCAB = 1000000
EMBED = 64
BATCH = 4096
SEQ = 200

NC, NS = 2, 16
NW = NC * NS
ROWS_PER_W = BATCH // NW
IDX_PER_W = ROWS_PER_W * SEQ
CHUNK = 8 * SEQ
NCHUNK = IDX_PER_W // CHUNK
NBUF = 4
NFULL = SEQ // 16
TAIL = SEQ - 16 * NFULL


@functools.partial(
    pl.kernel,
    mesh=plsc.VectorSubcoreMesh(core_axis_name="c", subcore_axis_name="s"),
    out_type=jax.ShapeDtypeStruct((BATCH, EMBED), jnp.float32),
    scratch_types=[
        pltpu.VMEM((ROWS_PER_W, EMBED), jnp.float32),
        pltpu.VMEM((CHUNK + 16,), jnp.int32),
    ] + [pltpu.VMEM((SEQ, EMBED // 2), jnp.int32) for _ in range(NBUF)]
      + [pltpu.SemaphoreType.DMA for _ in range(NBUF)],
)
def _sc_pool(idx_hbm, table_hbm, out_hbm, out_v, idx_v, *bufs_and_sems):
    bufs = bufs_and_sems[:NBUF]
    sems = bufs_and_sems[NBUF:]
    wid = lax.axis_index("s") * NC + lax.axis_index("c")
    base_i = wid * IDX_PER_W
    base_r = wid * ROWS_PER_W

    def load_chunk(c):
        pltpu.sync_copy(
            idx_hbm.at[pl.ds(base_i + c * CHUNK, CHUNK + 16)], idx_v)

    def issue16(woff, k, buf, sem, n=16):
        v = idx_v[pl.ds(woff + k * 16, 16)]
        for l in range(n):
            r = v[l]
            pltpu.async_copy(
                table_hbm.at[pl.ds(r, 1), :],
                buf.at[pl.ds(k * 16 + l, 1), :],
                sem)

    def issue_window(woff, b):
        def body(k, carry):
            issue16(woff, k, bufs[b], sems[b])
            return carry
        lax.fori_loop(0, NFULL, body, 0)
        issue16(woff, NFULL, bufs[b], sems[b], n=TAIL)

    def wait_window(b):
        pltpu.make_async_copy(
            table_hbm.at[pl.ds(0, SEQ), :], bufs[b], sems[b]).wait()

    def consume_window(win, b_red, b_iss):
        nxt = win + 3
        woff = lax.rem(nxt, 8) * SEQ
        do_issue = nxt < ROWS_PER_W

        @pl.when(jnp.logical_and(lax.rem(nxt, 8) == 0, do_issue))
        def _():
            load_chunk(nxt // 8)

        wait_window(b_red)
        buf = bufs[b_red]
        zero = jnp.zeros((16,), jnp.float32)

        def row_add(new, row):
            shamt = jnp.full((16,), 16, dtype=jnp.int32)
            msk = jnp.full((16,), -65536, dtype=jnp.int32)
            for c2 in range(EMBED // 32):
                v = buf[row, pl.ds(c2 * 16, 16)]
                lo = lax.bitcast_convert_type(
                    lax.shift_left(v, shamt), jnp.float32)
                hi = lax.bitcast_convert_type(
                    lax.bitwise_and(v, msk), jnp.float32)
                new[2 * c2] = new[2 * c2] + lo
                new[2 * c2 + 1] = new[2 * c2 + 1] + hi
            return new

        def body(k, accs):
            @pl.when(do_issue)
            def _():
                issue16(woff, k, bufs[b_iss], sems[b_iss])
            new = list(accs)
            for r in range(16):
                new = row_add(new, k * 16 + r)
            return tuple(new)

        accs = lax.fori_loop(0, NFULL, body, (zero,) * (EMBED // 16))

        @pl.when(do_issue)
        def _():
            issue16(woff, NFULL, bufs[b_iss], sems[b_iss], n=TAIL)
        accs = list(accs)
        for r in range(TAIL):
            accs = row_add(accs, NFULL * 16 + r)
        for c in range(EMBED // 16):
            out_v[win, pl.ds(c * 16, 16)] = accs[c]

    load_chunk(0)
    issue_window(0 * SEQ, 0)
    issue_window(1 * SEQ, 1)
    issue_window(2 * SEQ, 2)

    def step(i, carry):
        for b in range(NBUF):
            win = NBUF * i + b
            consume_window(win, b, (b + 3) % NBUF)
        return carry

    lax.fori_loop(0, ROWS_PER_W // NBUF, step, 0)

    pltpu.sync_copy(out_v, out_hbm.at[pl.ds(base_r, ROWS_PER_W)])


def _mlp_body(ps_ref, w1_ref, b1_ref, w2_ref, b2_ref, o_ref):
    pooled = ps_ref[...] * (1.0 / SEQ)
    h = jnp.dot(pooled, w1_ref[...], preferred_element_type=jnp.float32)
    h = jnp.maximum(h + b1_ref[...], 0.0)
    z = jnp.sum(h * w2_ref[...], axis=1, keepdims=True) + b2_ref[...]
    o_ref[...] = 1.0 / (1.0 + jnp.exp(-z))


def _mlp(pooled_sum, W1, b1, W2, b2):
    return pl.pallas_call(
        _mlp_body,
        out_shape=jax.ShapeDtypeStruct((BATCH, 1), jnp.float32),
    )(pooled_sum, W1, b1.reshape(1, EMBED), W2.reshape(1, EMBED),
      b2.reshape(1, 1))


def kernel(x, table, W1, b1, W2, b2):
    idx = x.astype(jnp.int32).reshape(-1)
    idx = jnp.concatenate([idx, jnp.arange(16, dtype=jnp.int32)])
    tb = jax.lax.bitcast_convert_type(
        table.astype(jnp.bfloat16).reshape(VOCAB, EMBED // 2, 2),
        jnp.int32)
    pooled_sum = _sc_pool(idx, tb)
    perm = jnp.concatenate([
        jnp.arange(0, 32, 2), jnp.arange(1, 32, 2),
        jnp.arange(32, 64, 2), jnp.arange(33, 64, 2)])
    return _mlp(pooled_sum, W1[perm, :], b1, W2, b2)


# speedup vs baseline: 3.1964x; 3.1964x over previous
import functools

import jax
import jax.numpy as jnp
from jax import lax
from jax.experimental import pallas as pl
from jax.experimental.pallas import tpu as pltpu
from jax.experimental.pallas import tpu_sc as plsc

VOCAB = 1000000
EMBED = 64
BATCH = 4096
SEQ = 200

NC, NS = 2, 16
NW = NC * NS
ROWS_PER_W = BATCH // NW
IDX_PER_W = ROWS_PER_W * SEQ
CHUNK = 8 * SEQ
NCHUNK = IDX_PER_W // CHUNK
NBUF = 4
NFULL = SEQ // 16
TAIL = SEQ - 16 * NFULL


@functools.partial(
    pl.kernel,
    mesh=plsc.VectorSubcoreMesh(core_axis_name="c", subcore_axis_name="s"),
    out_type=jax.ShapeDtypeStruct((BATCH, EMBED), jnp.float32),
    scratch_types=[
        pltpu.VMEM((ROWS_PER_W, EMBED), jnp.float32),
        pltpu.VMEM((CHUNK + 16,), jnp.int32),
    ] + [pltpu.VMEM((SEQ, EMBED), jnp.float32) for _ in range(NBUF)]
      + [pltpu.SemaphoreType.DMA for _ in range(NBUF)],
)
def _sc_pool(idx_hbm, table_hbm, out_hbm, out_v, idx_v, *bufs_and_sems):
    bufs = bufs_and_sems[:NBUF]
    sems = bufs_and_sems[NBUF:]
    wid = lax.axis_index("s") * NC + lax.axis_index("c")
    base_i = wid * IDX_PER_W
    base_r = wid * ROWS_PER_W

    def load_chunk(c):
        pltpu.sync_copy(
            idx_hbm.at[pl.ds(base_i + c * CHUNK, CHUNK + 16)], idx_v)

    def issue16(woff, k, buf, sem, n=16):
        v = idx_v[pl.ds(woff + k * 16, 16)]
        for l in range(n):
            r = v[l]
            pltpu.async_copy(
                table_hbm.at[pl.ds(r, 1), :],
                buf.at[pl.ds(k * 16 + l, 1), :],
                sem)

    def issue_window(woff, b):
        def body(k, carry):
            issue16(woff, k, bufs[b], sems[b])
            return carry
        lax.fori_loop(0, NFULL, body, 0)
        issue16(woff, NFULL, bufs[b], sems[b], n=TAIL)

    def wait_window(b):
        pltpu.make_async_copy(
            table_hbm.at[pl.ds(0, SEQ), :], bufs[b], sems[b]).wait()

    def consume_window(win, b_red, b_iss):
        nxt = win + 3
        woff = lax.rem(nxt, 8) * SEQ
        do_issue = nxt < ROWS_PER_W

        @pl.when(jnp.logical_and(lax.rem(nxt, 8) == 0, do_issue))
        def _():
            load_chunk(nxt // 8)

        wait_window(b_red)
        buf = bufs[b_red]
        zero = jnp.zeros((16,), jnp.float32)

        def body(k, accs):
            @pl.when(do_issue)
            def _():
                issue16(woff, k, bufs[b_iss], sems[b_iss])
            new = list(accs)
            for r in range(16):
                for c in range(EMBED // 16):
                    new[c] = new[c] + buf[k * 16 + r, pl.ds(c * 16, 16)]
            return tuple(new)

        accs = lax.fori_loop(0, NFULL, body, (zero,) * (EMBED // 16))

        @pl.when(do_issue)
        def _():
            issue16(woff, NFULL, bufs[b_iss], sems[b_iss], n=TAIL)
        accs = list(accs)
        for r in range(TAIL):
            for c in range(EMBED // 16):
                accs[c] = accs[c] + buf[NFULL * 16 + r, pl.ds(c * 16, 16)]
        for c in range(EMBED // 16):
            out_v[win, pl.ds(c * 16, 16)] = accs[c]

    load_chunk(0)
    issue_window(0 * SEQ, 0)
    issue_window(1 * SEQ, 1)
    issue_window(2 * SEQ, 2)

    def step(i, carry):
        for b in range(NBUF):
            win = NBUF * i + b
            consume_window(win, b, (b + 3) % NBUF)
        return carry

    lax.fori_loop(0, ROWS_PER_W // NBUF, step, 0)

    pltpu.sync_copy(out_v, out_hbm.at[pl.ds(base_r, ROWS_PER_W)])


def _mlp_body(ps_ref, w1_ref, b1_ref, w2_ref, b2_ref, o_ref):
    pooled = ps_ref[...] * (1.0 / SEQ)
    h = jnp.dot(pooled, w1_ref[...], preferred_element_type=jnp.float32)
    h = jnp.maximum(h + b1_ref[...], 0.0)
    z = jnp.sum(h * w2_ref[...], axis=1, keepdims=True) + b2_ref[...]
    o_ref[...] = 1.0 / (1.0 + jnp.exp(-z))


def _mlp(pooled_sum, W1, b1, W2, b2):
    return pl.pallas_call(
        _mlp_body,
        out_shape=jax.ShapeDtypeStruct((BATCH, 1), jnp.float32),
    )(pooled_sum, W1, b1.reshape(1, EMBED), W2.reshape(1, EMBED),
      b2.reshape(1, 1))


def kernel(x, table, W1, b1, W2, b2):
    idx = x.astype(jnp.int32).reshape(-1)
    idx = jnp.concatenate([idx, jnp.arange(16, dtype=jnp.int32)])
    pooled_sum = _sc_pool(idx, table)
    return _mlp(pooled_sum, W1, b1, W2, b2)
